# baseline (speedup 1.0000x reference)
import jax
import jax.numpy as jnp
from jax import lax
from jax.experimental import pallas as pl
from jax.experimental.pallas import tpu as pltpu
from jax.experimental.pallas import tpu_sc as plsc

MAX_SIZE = 38
NODE_DIM = 9
BOND_DIM = 4
N_EDGES = 378
B = 16384
NODE_W = MAX_SIZE * NODE_DIM
EDGE_W = N_EDGES * BOND_DIM
NODE_WP = 352

NW = 32
ECH = 8
BCH = B // ECH

_LN2 = 0.6931471805599453
_SQRTH = 0.7071067811865476
_C3 = 2.0 / 3.0
_C5 = 2.0 / 5.0
_C7 = 2.0 / 7.0


def _log(u):
    bits = lax.bitcast_convert_type(u, jnp.int32)
    e = (bits >> 23) - 126
    m = lax.bitcast_convert_type(
        (bits & 0x007FFFFF) | 0x3F000000, jnp.float32)
    cond = m < _SQRTH
    m = jnp.where(cond, m + m, m)
    ef = (e - cond.astype(jnp.int32)).astype(jnp.float32)
    r = (m - 1.0) / (m + 1.0)
    r2 = r * r
    w = ((_C7 * r2 + _C5) * r2 + _C3) * r2 + 2.0
    return ef * _LN2 + r * w


def _argmin_step(j, s, best, bj):
    lt = s < best
    return jnp.where(lt, s, best), jnp.where(lt, jnp.int32(j), bj)


def _make_body(n_pos, n_cat, width, unroll, rows, out_t):
    rows_w = rows // NW
    groups = rows_w // 16

    def body(ut, negc_t, lp_t, out_lp, out_oh,
             ub0, ub1, oh0, oh1, tnc, tnl, acc0, acc1, sems):
        wid = lax.axis_index("s") * 2 + lax.axis_index("c")

        pltpu.sync_copy(negc_t, tnc)
        pltpu.sync_copy(lp_t, tnl)

        lane = lax.iota(jnp.int32, 16)
        base = lane * width

        slots = ((ub0, oh0, acc0), (ub1, oh1, acc1))

        def in_copies(g, slot):
            ub = slots[slot][0]
            r0 = wid * rows_w + g * 16
            return (pltpu.make_async_copy(
                ut.at[pl.ds(r0 * width, 16 * width)], ub, sems.at[slot, 0]),)

        def out_copies(g, slot):
            _, oh, acc = slots[slot]
            r0 = wid * rows_w + g * 16
            return (pltpu.make_async_copy(
                        oh, out_oh.at[pl.ds(r0 * width, 16 * width)],
                        sems.at[slot, 1]),
                    pltpu.make_async_copy(
                        acc, out_lp.at[pl.ds(r0, 16)], sems.at[slot, 2]))

        def compute(g, slot):
            ub, oh, accb = slots[slot]

            def pos(p, acc):
                off = p * n_cat
                best = jnp.full((16,), jnp.float32(jnp.inf))
                bj = jnp.zeros((16,), jnp.int32)
                for j in range(n_cat):
                    u = plsc.load_gather(ub, [base + (j * n_pos + p)])
                    u = jnp.maximum(u, 1e-10)
                    negc = plsc.load_gather(
                        tnc, [jnp.full((16,), off + j, jnp.int32)])
                    s = _log(u) * negc
                    best, bj = _argmin_step(j, s, best, bj)
                ll = plsc.load_gather(tnl, [off + bj])
                for j in range(n_cat):
                    oidx = (j * n_pos + p) if out_t else (off + j)
                    plsc.store_scatter(oh, [base + oidx],
                                       (bj == j).astype(jnp.float32))
                return acc + ll

            acc = plsc.parallel_loop(0, n_pos, unroll=unroll,
                                     carry=jnp.zeros((16,), jnp.float32))(pos)
            accb[...] = acc

        for c in in_copies(0, 0):
            c.start()

        def pair(p, _):
            for k in range(2):
                g = p * 2 + k

                @pl.when(g + 1 < groups)
                def _():
                    for c in in_copies(g + 1, 1 - k):
                        c.start()

                for c in in_copies(g, k):
                    c.wait()

                @pl.when(g >= 2)
                def _():
                    for c in out_copies(g - 2, k):
                        c.wait()

                compute(g, k)
                for c in out_copies(g, k):
                    c.start()
            return 0

        lax.fori_loop(0, groups // 2, pair, 0)
        for c in out_copies(groups - 2, 0):
            c.wait()
        for c in out_copies(groups - 1, 1):
            c.wait()

    return body


def _make_call(n_pos, n_cat, width, width_p, unroll, rows, out_t=False):
    mesh = plsc.VectorSubcoreMesh(core_axis_name="c", subcore_axis_name="s")
    return pl.kernel(
        _make_body(n_pos, n_cat, width, unroll, rows, out_t),
        out_type=[
            jax.ShapeDtypeStruct((rows,), jnp.float32),
            jax.ShapeDtypeStruct((rows * width,), jnp.float32),
        ],
        mesh=mesh,
        compiler_params=pltpu.CompilerParams(needs_layout_passes=False),
        scratch_types=[
            pltpu.VMEM((16 * width,), jnp.float32),
            pltpu.VMEM((16 * width,), jnp.float32),
            pltpu.VMEM((16 * width,), jnp.float32),
            pltpu.VMEM((16 * width,), jnp.float32),
            pltpu.VMEM((width_p,), jnp.float32),
            pltpu.VMEM((width_p,), jnp.float32),
            pltpu.VMEM((16,), jnp.float32),
            pltpu.VMEM((16,), jnp.float32),
            pltpu.SemaphoreType.DMA((2, 3)),
        ],
    )


@jax.jit
def kernel(u_node, u_edge, node_base_log_probs, edge_base_log_probs):
    nl = node_base_log_probs * 0.3
    el = edge_base_log_probs / 0.3
    n_negc = jnp.pad(-jnp.exp(-nl).reshape(-1), (0, NODE_WP - NODE_W),
                     constant_values=-1.0)
    n_lp = jnp.pad(jax.nn.log_softmax(nl, axis=-1).reshape(-1),
                   (0, NODE_WP - NODE_W))
    e_negc = -jnp.exp(-el).reshape(-1)
    e_lp = jax.nn.log_softmax(el, axis=-1).reshape(-1)

    node_call = _make_call(MAX_SIZE, NODE_DIM, NODE_W, NODE_WP, 2, B, out_t=True)
    edge_call = _make_call(N_EDGES, BOND_DIM, EDGE_W, EDGE_W, 42, BCH, out_t=True)

    esums, eohs = [], []
    for ch in range(ECH):
        ue_t = jnp.swapaxes(u_edge[ch * BCH:(ch + 1) * BCH], 1, 2).reshape(-1)
        es, eo = edge_call(ue_t, e_negc, e_lp)
        esums.append(es)
        eohs.append(jnp.swapaxes(eo.reshape(BCH, BOND_DIM, N_EDGES), 1, 2))

    un_t = jnp.swapaxes(u_node, 1, 2).reshape(-1)
    nsum, noh = node_call(un_t, n_negc, n_lp)

    return (nsum + jnp.concatenate(esums),
            jnp.swapaxes(noh.reshape(B, NODE_DIM, MAX_SIZE), 1, 2),
            jnp.concatenate(eohs, axis=0))


# speedup vs baseline: 1.7202x; 1.7202x over previous
import jax
import jax.numpy as jnp
from jax import lax
from jax.experimental import pallas as pl
from jax.experimental.pallas import tpu as pltpu
from jax.experimental.pallas import tpu_sc as plsc

MAX_SIZE = 38
NODE_DIM = 9
BOND_DIM = 4
N_EDGES = 378
B = 16384
NODE_W = MAX_SIZE * NODE_DIM
EDGE_W = N_EDGES * BOND_DIM
NODE_WP = 352

NW = 32
ECH = 8
BCH = B // ECH

_LN2 = 0.6931471805599453
_SQRTH = 0.7071067811865476
_C3 = 2.0 / 3.0
_C5 = 2.0 / 5.0
_C7 = 2.0 / 7.0


def _log(u):
    bits = lax.bitcast_convert_type(u, jnp.int32)
    e = (bits >> 23) - 126
    m = lax.bitcast_convert_type(
        (bits & 0x007FFFFF) | 0x3F000000, jnp.float32)
    cond = m < _SQRTH
    m = jnp.where(cond, m + m, m)
    ef = (e - cond.astype(jnp.int32)).astype(jnp.float32)
    r = (m - 1.0) / (m + 1.0)
    r2 = r * r
    w = ((_C7 * r2 + _C5) * r2 + _C3) * r2 + 2.0
    return ef * _LN2 + r * w


def _argmin_step(j, s, best, bj):
    lt = s < best
    return jnp.where(lt, s, best), jnp.where(lt, jnp.int32(j), bj)


def _make_body(n_pos, n_cat, width, unroll, rows, out_t):
    rows_w = rows // NW
    groups = rows_w // 16

    def body(ut, negc_t, lp_t, out_lp, out_oh,
             ub0, ub1, oh0, oh1, tnc, tnl, acc0, acc1, sems):
        wid = lax.axis_index("s") * 2 + lax.axis_index("c")

        pltpu.sync_copy(negc_t, tnc)
        pltpu.sync_copy(lp_t, tnl)

        lane = lax.iota(jnp.int32, 16)
        base = lane * width

        slots = ((ub0, oh0, acc0), (ub1, oh1, acc1))

        def in_copies(g, slot):
            ub = slots[slot][0]
            r0 = wid * rows_w + g * 16
            return (pltpu.make_async_copy(
                ut.at[pl.ds(r0 * width, 16 * width)], ub, sems.at[slot, 0]),)

        def out_copies(g, slot):
            _, oh, acc = slots[slot]
            r0 = wid * rows_w + g * 16
            return (pltpu.make_async_copy(
                        oh, out_oh.at[pl.ds(r0 * width, 16 * width)],
                        sems.at[slot, 1]),
                    pltpu.make_async_copy(
                        acc, out_lp.at[pl.ds(r0, 16)], sems.at[slot, 2]))

        def compute(g, slot):
            ub, oh, accb = slots[slot]

            def pos(p, acc):
                off = p * n_cat
                best = jnp.full((16,), jnp.float32(jnp.inf))
                bj = jnp.zeros((16,), jnp.int32)
                for j in range(n_cat):
                    u = plsc.load_gather(ub, [base + (j * n_pos + p)])
                    u = jnp.maximum(u, 1e-10)
                    negc = plsc.load_gather(
                        tnc, [jnp.full((16,), off + j, jnp.int32)])
                    s = _log(u) * negc
                    best, bj = _argmin_step(j, s, best, bj)
                ll = plsc.load_gather(tnl, [off + bj])
                for j in range(n_cat):
                    oidx = (j * n_pos + p) if out_t else (off + j)
                    plsc.store_scatter(oh, [base + oidx],
                                       (bj == j).astype(jnp.float32))
                return acc + ll

            acc = plsc.parallel_loop(0, n_pos, unroll=unroll,
                                     carry=jnp.zeros((16,), jnp.float32))(pos)
            accb[...] = acc

        for c in in_copies(0, 0):
            c.start()

        def pair(p, _):
            for k in range(2):
                g = p * 2 + k

                @pl.when(g + 1 < groups)
                def _():
                    for c in in_copies(g + 1, 1 - k):
                        c.start()

                for c in in_copies(g, k):
                    c.wait()

                @pl.when(g >= 2)
                def _():
                    for c in out_copies(g - 2, k):
                        c.wait()

                compute(g, k)
                for c in out_copies(g, k):
                    c.start()
            return 0

        lax.fori_loop(0, groups // 2, pair, 0)
        for c in out_copies(groups - 2, 0):
            c.wait()
        for c in out_copies(groups - 1, 1):
            c.wait()

    return body


def _make_call(n_pos, n_cat, width, width_p, unroll, rows, out_t=False):
    mesh = plsc.VectorSubcoreMesh(core_axis_name="c", subcore_axis_name="s")
    return pl.kernel(
        _make_body(n_pos, n_cat, width, unroll, rows, out_t),
        out_type=[
            jax.ShapeDtypeStruct((rows,), jnp.float32),
            jax.ShapeDtypeStruct((rows * width,), jnp.float32),
        ],
        mesh=mesh,
        compiler_params=pltpu.CompilerParams(needs_layout_passes=False),
        scratch_types=[
            pltpu.VMEM((16 * width,), jnp.float32),
            pltpu.VMEM((16 * width,), jnp.float32),
            pltpu.VMEM((16 * width,), jnp.float32),
            pltpu.VMEM((16 * width,), jnp.float32),
            pltpu.VMEM((width_p,), jnp.float32),
            pltpu.VMEM((width_p,), jnp.float32),
            pltpu.VMEM((16,), jnp.float32),
            pltpu.VMEM((16,), jnp.float32),
            pltpu.SemaphoreType.DMA((2, 3)),
        ],
    )


@jax.jit
def kernel(u_node, u_edge, node_base_log_probs, edge_base_log_probs):
    nl = node_base_log_probs * 0.3
    el = edge_base_log_probs / 0.3
    n_negc = jnp.pad(-jnp.exp(-nl).reshape(-1), (0, NODE_WP - NODE_W),
                     constant_values=-1.0)
    n_lp = jnp.pad(jax.nn.log_softmax(nl, axis=-1).reshape(-1),
                   (0, NODE_WP - NODE_W))
    e_negc = -jnp.exp(-el).reshape(-1)
    e_lp = jax.nn.log_softmax(el, axis=-1).reshape(-1)

    node_call = _make_call(MAX_SIZE, NODE_DIM, NODE_W, NODE_WP, 2, B, out_t=True)
    edge_call = _make_call(N_EDGES, BOND_DIM, EDGE_W, EDGE_W, 21, BCH, out_t=True)

    esums, eohs = [], []
    for ch in range(ECH):
        ue_t = jnp.swapaxes(u_edge[ch * BCH:(ch + 1) * BCH], 1, 2).reshape(-1)
        es, eo = edge_call(ue_t, e_negc, e_lp)
        esums.append(es)
        eohs.append(jnp.swapaxes(eo.reshape(BCH, BOND_DIM, N_EDGES), 1, 2))

    un_t = jnp.swapaxes(u_node, 1, 2).reshape(-1)
    nsum, noh = node_call(un_t, n_negc, n_lp)

    return (nsum + jnp.concatenate(esums),
            jnp.swapaxes(noh.reshape(B, NODE_DIM, MAX_SIZE), 1, 2),
            jnp.concatenate(eohs, axis=0))
